# baseline (speedup 1.0000x reference)
import jax
import jax.numpy as jnp
from jax.experimental import pallas as pl
from jax.experimental.pallas import tpu as pltpu

B, S, DIM, E, K = 2, 32, 512, 64, 2
H = 4 * DIM
T = B * S
NB = 3

_SQRT_HALF = 0.7071067811865476


def _gelu(t):
    return 0.5 * t * (1.0 + jax.lax.erf(t * _SQRT_HALF))


def _moe_kernel(x_ref, gw1_hbm, gw2_hbm, gw3_hbm, gb1_ref, gb2_ref, gb3_ref,
                eb1_ref, eb2_ref, ew1_hbm, ew2_hbm, out_ref,
                gw1_v, gw2_v, gw3_v, w1buf, w2buf, gsem, w1sem, w2sem):
    hi = None
    xt = x_ref[...]

    def gw_copy(i, src, dst):
        return pltpu.make_async_copy(src, dst, gsem.at[i])

    def w_copy(e, slot):
        return (pltpu.make_async_copy(ew1_hbm.at[e], w1buf.at[slot],
                                      w1sem.at[slot]),
                pltpu.make_async_copy(ew2_hbm.at[e], w2buf.at[slot],
                                      w2sem.at[slot]))

    gw_copy(0, gw1_hbm, gw1_v).start()
    gw_copy(1, gw2_hbm, gw2_v).start()
    gw_copy(2, gw3_hbm, gw3_v).start()
    for e0 in range(NB):
        c1, c2 = w_copy(e0, e0)
        c1.start()
        c2.start()

    gw_copy(0, gw1_hbm, gw1_v).wait()
    gw_copy(1, gw2_hbm, gw2_v).wait()
    gw_copy(2, gw3_hbm, gw3_v).wait()

    g = _gelu(jnp.dot(xt, gw1_v[...], precision=hi,
                      preferred_element_type=jnp.float32) + gb1_ref[0])
    g = _gelu(jnp.dot(g, gw2_v[...], precision=hi,
                      preferred_element_type=jnp.float32) + gb2_ref[0])
    logits = jax.nn.sigmoid(jnp.dot(g, gw3_v[...], precision=hi,
                                    preferred_element_type=jnp.float32)
                            + gb3_ref[0])
    iota = jax.lax.broadcasted_iota(jnp.int32, (T, E), 1)
    v1 = jnp.max(logits, axis=1, keepdims=True)
    i1 = jnp.min(jnp.where(logits == v1, iota, E), axis=1, keepdims=True)
    masked = jnp.where(iota == i1, -jnp.inf, logits)
    v2 = jnp.max(masked, axis=1, keepdims=True)
    i2 = jnp.min(jnp.where(masked == v2, iota, E), axis=1, keepdims=True)
    s = v1 + v2
    v1n = v1 / s
    v2n = v2 / s

    out_ref[...] = jnp.zeros((T, DIM), jnp.float32)

    def body(e, carry):
        slot = jax.lax.rem(e, NB)
        c1, c2 = w_copy(e, slot)
        c1.wait()
        c2.wait()

        @pl.when(e + NB < E)
        def _prefetch():
            n1, n2 = w_copy(e + NB, slot)
            n1.start()
            n2.start()

        h = _gelu(jax.lax.dot_general(xt, w1buf[slot],
                                      (((1,), (1,)), ((), ())),
                                      precision=hi,
                                      preferred_element_type=jnp.float32)
                  + eb1_ref[e])
        o = _gelu(jnp.dot(h, w2buf[slot], precision=hi,
                          preferred_element_type=jnp.float32) + eb2_ref[e])
        scale = (jnp.where(i1 == e, v1n, 0.0)
                 + jnp.where(i2 == e, v2n, 0.0))
        out_ref[...] += scale * o
        return carry

    jax.lax.fori_loop(0, E, body, 0)


def kernel(x, gw1, gb1, gw2, gb2, gw3, gb3, ew1, ew2, eb1, eb2):
    xt = x.reshape(T, DIM)
    eb1r = eb1.reshape(E, 1, H)
    eb2r = eb2.reshape(E, 1, DIM)

    out = pl.pallas_call(
        _moe_kernel,
        in_specs=[
            pl.BlockSpec((T, DIM), lambda: (0, 0)),
            pl.BlockSpec(memory_space=pl.ANY),
            pl.BlockSpec(memory_space=pl.ANY),
            pl.BlockSpec(memory_space=pl.ANY),
            pl.BlockSpec((1, H), lambda: (0, 0)),
            pl.BlockSpec((1, H), lambda: (0, 0)),
            pl.BlockSpec((1, E), lambda: (0, 0)),
            pl.BlockSpec((E, 1, H), lambda: (0, 0, 0)),
            pl.BlockSpec((E, 1, DIM), lambda: (0, 0, 0)),
            pl.BlockSpec(memory_space=pl.ANY),
            pl.BlockSpec(memory_space=pl.ANY),
        ],
        out_specs=pl.BlockSpec((T, DIM), lambda: (0, 0)),
        out_shape=jax.ShapeDtypeStruct((T, DIM), jnp.float32),
        scratch_shapes=[
            pltpu.VMEM((DIM, H), jnp.float32),
            pltpu.VMEM((H, H), jnp.float32),
            pltpu.VMEM((H, E), jnp.float32),
            pltpu.VMEM((NB, H, DIM), jnp.float32),
            pltpu.VMEM((NB, H, DIM), jnp.float32),
            pltpu.SemaphoreType.DMA((3,)),
            pltpu.SemaphoreType.DMA((NB,)),
            pltpu.SemaphoreType.DMA((NB,)),
        ],
    )(xt, gw1, gw2, gw3, gb1.reshape(1, H), gb2.reshape(1, H),
      gb3.reshape(1, E), eb1r, eb2r, ew1, ew2)

    return out.reshape(B, S, DIM)
